# Initial kernel scaffold; baseline (speedup 1.0000x reference)
import jax
import jax.numpy as jnp
from jax.experimental import pallas as pl


def kernel(x, edge_index, W_gcn, b_gcn, W_fc, b_fc):
    raise NotImplementedError("write your pallas kernel here")


# speedup vs baseline: 3.0658x; 3.0658x over previous
import functools

import jax
import jax.numpy as jnp
from jax import lax
from jax.experimental import pallas as pl
from jax.experimental.pallas import tpu as pltpu
from jax.experimental.pallas import tpu_sc as plsc

N_NODES = 10000
FEAT = 256
N_EDGES = 160000
OUT_C = 2

NC = 2
NS = 16
NW = NC * NS
CHUNK = 128
E_PAD = 163840
ROWS2D = E_PAD // CHUNK
GRP = 40
NGRP = ROWS2D // GRP
PAD_DST = 1 << 29

WIN = 320
DUMP = WIN
ACC_ROWS = 328
DEG_PAD = 384

_MESH = plsc.VectorSubcoreMesh(
    core_axis_name="c", subcore_axis_name="s", num_cores=NC, num_subcores=NS)

_Z16F = functools.partial(jnp.zeros, (16,), jnp.float32)
_SC_PARAMS = pltpu.CompilerParams(needs_layout_passes=False)


def _build_deg(interpret=False):
    @functools.partial(
        pl.kernel,
        out_type=jax.ShapeDtypeStruct((NW, DEG_PAD), jnp.float32),
        mesh=_MESH,
        scratch_types=[
            pltpu.VMEM((GRP, CHUNK), jnp.int32),
            pltpu.VMEM((DEG_PAD,), jnp.float32),
        ],
        compiler_params=_SC_PARAMS,
        interpret=interpret,
    )
    def deg_kernel(dst_hbm, pdeg_hbm, dstv, acc):
        c = lax.axis_index("c")
        s = lax.axis_index("s")
        w = s * NC + c
        lo = w * WIN

        def zacc(i, _):
            acc[pl.ds(pl.multiple_of(i * 16, 16), 16)] = _Z16F()
            return 0
        lax.fori_loop(0, DEG_PAD // 16, zacc, 0)
        ones16 = jnp.ones((16,), jnp.float32)

        def grp(g, _):
            pltpu.sync_copy(dst_hbm.at[pl.ds(g * GRP, GRP)], dstv)

            def row(r, _):
                for u in range(CHUNK // 16):
                    d = dstv[r, pl.ds(u * 16, 16)]
                    m = (d >= lo) & (d < lo + WIN)
                    plsc.addupdate_scatter(acc, [d - lo], ones16, mask=m)
                return 0
            lax.fori_loop(0, GRP, row, 0)
            return 0
        lax.fori_loop(0, NGRP, grp, 0)

        pltpu.sync_copy(acc, pdeg_hbm.at[w])

    return deg_kernel


def _build_msg(interpret=False):
    @functools.partial(
        pl.kernel,
        out_type=jax.ShapeDtypeStruct((NW * WIN, FEAT), jnp.float32),
        mesh=_MESH,
        scratch_types=[
            pltpu.VMEM((GRP, CHUNK), jnp.int32),
            pltpu.VMEM((GRP, CHUNK), jnp.int32),
            pltpu.VMEM((CHUNK,), jnp.int32),
            pltpu.VMEM((CHUNK,), jnp.int32),
            pltpu.VMEM((CHUNK, FEAT), jnp.float32),
            pltpu.VMEM((ACC_ROWS, FEAT), jnp.float32),
            pltpu.SemaphoreType.DMA,
        ],
        compiler_params=_SC_PARAMS,
        interpret=interpret,
    )
    def msg_kernel(hs_hbm, src_hbm, dst_hbm, agg_hbm,
                   srcv, dstv, sels, seld, rows, acc, sem):
        c = lax.axis_index("c")
        s = lax.axis_index("s")
        w = s * NC + c
        lo = w * WIN
        iota = lax.iota(jnp.int32, 16)

        def zacc(i, _):
            for u in range(FEAT // 16):
                acc[i, pl.ds(u * 16, 16)] = _Z16F()
            return 0
        lax.fori_loop(0, ACC_ROWS, zacc, 0)
        for u in range(CHUNK // 16):
            seld[pl.ds(u * 16, 16)] = jnp.full((16,), DUMP, jnp.int32)
            sels[pl.ds(u * 16, 16)] = jnp.zeros((16,), jnp.int32)

        def flush(off):
            pltpu.async_copy(hs_hbm.at[sels], rows, sem).wait()

            def addrow(j, _):
                base = pl.multiple_of((j // 16) * 16, 16)
                lane = j - base
                vec = seld[pl.ds(base, 16)]
                dj = jnp.sum(jnp.where(iota == lane, vec, 0))
                for u in range(FEAT // 16):
                    plsc.addupdate(acc.at[dj, pl.ds(u * 16, 16)],
                                   rows[j, pl.ds(u * 16, 16)])
                return 0
            lax.fori_loop(0, CHUNK, addrow, 0)
            for u in range(CHUNK // 16):
                seld[pl.ds(u * 16, 16)] = jnp.full((16,), DUMP, jnp.int32)
            return jnp.int32(0)

        def grp(g, off):
            pltpu.sync_copy(src_hbm.at[pl.ds(g * GRP, GRP)], srcv)
            pltpu.sync_copy(dst_hbm.at[pl.ds(g * GRP, GRP)], dstv)

            def row(r, off):
                for u in range(CHUNK // 16):
                    d = dstv[r, pl.ds(u * 16, 16)]
                    sv = srcv[r, pl.ds(u * 16, 16)]
                    m = (d >= lo) & (d < lo + WIN)
                    plsc.store_compressed(seld.at[pl.ds(off, 16)], d - lo, mask=m)
                    plsc.store_compressed(sels.at[pl.ds(off, 16)], sv, mask=m)
                    off = off + jnp.sum(m.astype(jnp.int32))
                    off = lax.cond(off >= CHUNK - 16, flush, lambda o: o, off)
                return off
            return lax.fori_loop(0, GRP, row, off)
        off = lax.fori_loop(0, NGRP, grp, jnp.int32(0))
        off = lax.cond(off > 0, flush, lambda o: o, off)

        pltpu.sync_copy(acc.at[pl.ds(0, WIN)],
                        agg_hbm.at[pl.ds(w * WIN, WIN)])

    return msg_kernel


_ROW_BLK = 2000
_GRID = N_NODES // _ROW_BLK


def _gcn1_body(x_ref, w_ref, deg_ref, hs_ref):
    dinv = lax.rsqrt(deg_ref[...] + 1.0)
    h = jnp.dot(x_ref[...], w_ref[...], preferred_element_type=jnp.float32)
    hs_ref[...] = h * dinv


def _build_gcn1(interpret=False):
    return pl.pallas_call(
        _gcn1_body,
        grid=(_GRID,),
        in_specs=[
            pl.BlockSpec((_ROW_BLK, FEAT), lambda i: (i, 0)),
            pl.BlockSpec((FEAT, FEAT), lambda i: (0, 0)),
            pl.BlockSpec((_ROW_BLK, 1), lambda i: (i, 0)),
        ],
        out_specs=pl.BlockSpec((_ROW_BLK, FEAT), lambda i: (i, 0)),
        out_shape=jax.ShapeDtypeStruct((N_NODES, FEAT), jnp.float32),
        interpret=interpret,
    )


def _out_body(agg_ref, hs_ref, deg_ref, bg_ref, wfc_ref, bfc_ref, o_ref):
    dinv = lax.rsqrt(deg_ref[...] + 1.0)
    out = dinv * (agg_ref[...] + hs_ref[...]) + bg_ref[...]
    h2 = jnp.maximum(out, 0.0)
    lp = jnp.dot(h2, wfc_ref[...], preferred_element_type=jnp.float32) + bfc_ref[...]
    l0 = lp[:, 0:1]
    l1 = lp[:, 1:2]
    p0 = 1.0 / (1.0 + jnp.exp(l1 - l0))
    p1 = 1.0 / (1.0 + jnp.exp(l0 - l1))
    o_ref[...] = jnp.concatenate([p0, p1], axis=1)


def _build_out(interpret=False):
    return pl.pallas_call(
        _out_body,
        grid=(_GRID,),
        in_specs=[
            pl.BlockSpec((_ROW_BLK, FEAT), lambda i: (i, 0)),
            pl.BlockSpec((_ROW_BLK, FEAT), lambda i: (i, 0)),
            pl.BlockSpec((_ROW_BLK, 1), lambda i: (i, 0)),
            pl.BlockSpec((1, FEAT), lambda i: (0, 0)),
            pl.BlockSpec((FEAT, 128), lambda i: (0, 0)),
            pl.BlockSpec((1, 128), lambda i: (0, 0)),
        ],
        out_specs=pl.BlockSpec((_ROW_BLK, OUT_C), lambda i: (i, 0)),
        out_shape=jax.ShapeDtypeStruct((N_NODES, OUT_C), jnp.float32),
        interpret=interpret,
    )


_deg_kernel = _build_deg()
_msg_kernel = _build_msg()
_gcn1_kernel = _build_gcn1()
_out_kernel = _build_out()


def kernel(x, edge_index, W_gcn, b_gcn, W_fc, b_fc):
    src = edge_index[0].astype(jnp.int32)
    dst = edge_index[1].astype(jnp.int32)
    pad = E_PAD - N_EDGES
    src2d = jnp.concatenate([src, jnp.zeros((pad,), jnp.int32)]).reshape(ROWS2D, CHUNK)
    dst2d = jnp.concatenate([dst, jnp.full((pad,), PAD_DST, jnp.int32)]).reshape(ROWS2D, CHUNK)

    pdeg = _deg_kernel(dst2d)
    deg = pdeg[:, :WIN].reshape(-1)[:N_NODES, None]
    x2 = x.reshape(N_NODES, FEAT)
    hs = _gcn1_kernel(x2, W_gcn, deg)
    agg = _msg_kernel(hs, src2d, dst2d)

    wfc_p = jnp.pad(W_fc, ((0, 0), (0, 128 - OUT_C)))
    bfc_p = jnp.pad(b_fc, (0, 128 - OUT_C)).reshape(1, 128)
    probs = _out_kernel(agg, hs, deg, b_gcn.reshape(1, FEAT), wfc_p, bfc_p)
    return probs.reshape(1, N_NODES, OUT_C)


# speedup vs baseline: 5.5413x; 1.8075x over previous
import functools

import jax
import jax.numpy as jnp
from jax import lax
from jax.experimental import pallas as pl
from jax.experimental.pallas import tpu as pltpu
from jax.experimental.pallas import tpu_sc as plsc

N_NODES = 10000
FEAT = 256
N_EDGES = 160000
OUT_C = 2

NC = 2
NS = 16
NW = NC * NS
CHUNK = 128
E_PAD = 163840
ROWS2D = E_PAD // CHUNK
GRP = 40
NGRP = ROWS2D // GRP
PAD_DST = 1 << 29

WIN = 320
DUMP = WIN
ACC_ROWS = 328
DEG_PAD = 384

_MESH = plsc.VectorSubcoreMesh(
    core_axis_name="c", subcore_axis_name="s", num_cores=NC, num_subcores=NS)

_Z16F = functools.partial(jnp.zeros, (16,), jnp.float32)
_SC_PARAMS = pltpu.CompilerParams(needs_layout_passes=False)


def _build_deg(interpret=False):
    @functools.partial(
        pl.kernel,
        out_type=jax.ShapeDtypeStruct((NW, DEG_PAD), jnp.float32),
        mesh=_MESH,
        scratch_types=[
            pltpu.VMEM((GRP, CHUNK), jnp.int32),
            pltpu.VMEM((DEG_PAD,), jnp.float32),
        ],
        compiler_params=_SC_PARAMS,
        interpret=interpret,
    )
    def deg_kernel(dst_hbm, pdeg_hbm, dstv, acc):
        c = lax.axis_index("c")
        s = lax.axis_index("s")
        w = s * NC + c
        lo = w * WIN

        def zacc(i, _):
            acc[pl.ds(pl.multiple_of(i * 16, 16), 16)] = _Z16F()
            return 0
        lax.fori_loop(0, DEG_PAD // 16, zacc, 0)
        ones16 = jnp.ones((16,), jnp.float32)

        def grp(g, _):
            pltpu.sync_copy(dst_hbm.at[pl.ds(g * GRP, GRP)], dstv)

            def row(r, _):
                for u in range(CHUNK // 16):
                    d = dstv[r, pl.ds(u * 16, 16)]
                    m = (d >= lo) & (d < lo + WIN)
                    plsc.addupdate_scatter(acc, [d - lo], ones16, mask=m)
                return 0
            lax.fori_loop(0, GRP, row, 0)
            return 0
        lax.fori_loop(0, NGRP, grp, 0)

        pltpu.sync_copy(acc, pdeg_hbm.at[w])

    return deg_kernel


def _build_msg(interpret=False):
    @functools.partial(
        pl.kernel,
        out_type=jax.ShapeDtypeStruct((NW * WIN, FEAT), jnp.float32),
        mesh=_MESH,
        scratch_types=[
            pltpu.VMEM((GRP, CHUNK), jnp.int32),
            pltpu.VMEM((GRP, CHUNK), jnp.int32),
            pltpu.VMEM((2 * CHUNK,), jnp.int32),
            pltpu.VMEM((2 * CHUNK,), jnp.int32),
            pltpu.VMEM((CHUNK, FEAT), jnp.float32),
            pltpu.VMEM((ACC_ROWS, FEAT), jnp.float32),
            pltpu.SemaphoreType.DMA,
            pltpu.SemaphoreType.DMA,
        ],
        compiler_params=_SC_PARAMS,
        interpret=interpret,
    )
    def msg_kernel(hs_hbm, src_hbm, dst_hbm, agg_hbm,
                   srcv, dstv, sels, seld, rows, acc, sem, sem2):
        c = lax.axis_index("c")
        s = lax.axis_index("s")
        w = s * NC + c
        lo = w * WIN
        iota = lax.iota(jnp.int32, 16)

        def zacc(i, _):
            for u in range(FEAT // 16):
                acc[i, pl.ds(u * 16, 16)] = _Z16F()
            return 0
        lax.fori_loop(0, ACC_ROWS, zacc, 0)
        for u in range(2 * CHUNK // 16):
            seld[pl.ds(u * 16, 16)] = jnp.full((16,), DUMP, jnp.int32)
            sels[pl.ds(u * 16, 16)] = jnp.zeros((16,), jnp.int32)

        def flush(off):
            pltpu.async_copy(hs_hbm.at[sels.at[pl.ds(0, CHUNK)]], rows, sem).wait()

            def addgrp(g, _):
                vec = seld[pl.ds(pl.multiple_of(g * 16, 16), 16)]
                for l in range(16):
                    dj = jnp.sum(jnp.where(iota == l, vec, 0))
                    j = g * 16 + l
                    for u in range(FEAT // 16):
                        plsc.addupdate(acc.at[dj, pl.ds(u * 16, 16)],
                                       rows[j, pl.ds(u * 16, 16)])
                return 0
            lax.fori_loop(0, CHUNK // 16, addgrp, 0)
            for u in range(CHUNK // 16):
                seld[pl.ds(u * 16, 16)] = seld[pl.ds(CHUNK + u * 16, 16)]
                sels[pl.ds(u * 16, 16)] = sels[pl.ds(CHUNK + u * 16, 16)]
                seld[pl.ds(CHUNK + u * 16, 16)] = jnp.full((16,), DUMP, jnp.int32)
            return off - CHUNK

        def grp(g, off):
            h1 = pltpu.async_copy(src_hbm.at[pl.ds(g * GRP, GRP)], srcv, sem)
            h2 = pltpu.async_copy(dst_hbm.at[pl.ds(g * GRP, GRP)], dstv, sem2)
            h1.wait()
            h2.wait()

            def row(r, off):
                for u in range(CHUNK // 16):
                    d = dstv[r, pl.ds(u * 16, 16)]
                    sv = srcv[r, pl.ds(u * 16, 16)]
                    m = (d >= lo) & (d < lo + WIN)
                    plsc.store_compressed(seld.at[pl.ds(off, 16)], d - lo, mask=m)
                    plsc.store_compressed(sels.at[pl.ds(off, 16)], sv, mask=m)
                    off = off + jnp.sum(m.astype(jnp.int32))
                return lax.cond(off >= CHUNK, flush, lambda o: o, off)
            return lax.fori_loop(0, GRP, row, off)
        off = lax.fori_loop(0, NGRP, grp, jnp.int32(0))
        off = lax.cond(off > 0, lambda o: flush(o), lambda o: o, off)

        pltpu.sync_copy(acc.at[pl.ds(0, WIN)],
                        agg_hbm.at[pl.ds(w * WIN, WIN)])

    return msg_kernel


_ROW_BLK = 2000
_GRID = N_NODES // _ROW_BLK


def _gcn1_body(x_ref, w_ref, deg_ref, hs_ref):
    dinv = lax.rsqrt(deg_ref[...] + 1.0)
    h = jnp.dot(x_ref[...], w_ref[...], preferred_element_type=jnp.float32)
    hs_ref[...] = h * dinv


def _build_gcn1(interpret=False):
    return pl.pallas_call(
        _gcn1_body,
        grid=(_GRID,),
        in_specs=[
            pl.BlockSpec((_ROW_BLK, FEAT), lambda i: (i, 0)),
            pl.BlockSpec((FEAT, FEAT), lambda i: (0, 0)),
            pl.BlockSpec((_ROW_BLK, 1), lambda i: (i, 0)),
        ],
        out_specs=pl.BlockSpec((_ROW_BLK, FEAT), lambda i: (i, 0)),
        out_shape=jax.ShapeDtypeStruct((N_NODES, FEAT), jnp.float32),
        interpret=interpret,
    )


def _out_body(agg_ref, hs_ref, deg_ref, bg_ref, wfc_ref, bfc_ref, o_ref):
    dinv = lax.rsqrt(deg_ref[...] + 1.0)
    out = dinv * (agg_ref[...] + hs_ref[...]) + bg_ref[...]
    h2 = jnp.maximum(out, 0.0)
    lp = jnp.dot(h2, wfc_ref[...], preferred_element_type=jnp.float32) + bfc_ref[...]
    l0 = lp[:, 0:1]
    l1 = lp[:, 1:2]
    p0 = 1.0 / (1.0 + jnp.exp(l1 - l0))
    p1 = 1.0 / (1.0 + jnp.exp(l0 - l1))
    o_ref[...] = jnp.concatenate([p0, p1], axis=1)


def _build_out(interpret=False):
    return pl.pallas_call(
        _out_body,
        grid=(_GRID,),
        in_specs=[
            pl.BlockSpec((_ROW_BLK, FEAT), lambda i: (i, 0)),
            pl.BlockSpec((_ROW_BLK, FEAT), lambda i: (i, 0)),
            pl.BlockSpec((_ROW_BLK, 1), lambda i: (i, 0)),
            pl.BlockSpec((1, FEAT), lambda i: (0, 0)),
            pl.BlockSpec((FEAT, 128), lambda i: (0, 0)),
            pl.BlockSpec((1, 128), lambda i: (0, 0)),
        ],
        out_specs=pl.BlockSpec((_ROW_BLK, OUT_C), lambda i: (i, 0)),
        out_shape=jax.ShapeDtypeStruct((N_NODES, OUT_C), jnp.float32),
        interpret=interpret,
    )


_deg_kernel = _build_deg()
_msg_kernel = _build_msg()
_gcn1_kernel = _build_gcn1()
_out_kernel = _build_out()


def kernel(x, edge_index, W_gcn, b_gcn, W_fc, b_fc):
    src = edge_index[0].astype(jnp.int32)
    dst = edge_index[1].astype(jnp.int32)
    pad = E_PAD - N_EDGES
    src2d = jnp.concatenate([src, jnp.zeros((pad,), jnp.int32)]).reshape(ROWS2D, CHUNK)
    dst2d = jnp.concatenate([dst, jnp.full((pad,), PAD_DST, jnp.int32)]).reshape(ROWS2D, CHUNK)

    pdeg = _deg_kernel(dst2d)
    deg = pdeg[:, :WIN].reshape(-1)[:N_NODES, None]
    x2 = x.reshape(N_NODES, FEAT)
    hs = _gcn1_kernel(x2, W_gcn, deg)
    agg = _msg_kernel(hs, src2d, dst2d)

    wfc_p = jnp.pad(W_fc, ((0, 0), (0, 128 - OUT_C)))
    bfc_p = jnp.pad(b_fc, (0, 128 - OUT_C)).reshape(1, 128)
    probs = _out_kernel(agg, hs, deg, b_gcn.reshape(1, FEAT), wfc_p, bfc_p)
    return probs.reshape(1, N_NODES, OUT_C)


# speedup vs baseline: 6.3467x; 1.1453x over previous
import functools

import jax
import jax.numpy as jnp
from jax import lax
from jax.experimental import pallas as pl
from jax.experimental.pallas import tpu as pltpu
from jax.experimental.pallas import tpu_sc as plsc

N_NODES = 10000
FEAT = 256
N_EDGES = 160000
OUT_C = 2

NC = 2
NS = 16
NW = NC * NS
CHUNK = 128
E_PAD = 163840
ROWS2D = E_PAD // CHUNK
GRP = 40
NGRP = ROWS2D // GRP
PAD_DST = 1 << 29

WIN = 320
DUMP = WIN
ACC_ROWS = 328
HIST = 10368
HDUMP = 10240

_MESH = plsc.VectorSubcoreMesh(
    core_axis_name="c", subcore_axis_name="s", num_cores=NC, num_subcores=NS)

_Z16F = functools.partial(jnp.zeros, (16,), jnp.float32)
_SC_PARAMS = pltpu.CompilerParams(needs_layout_passes=False)


def _build_deg(interpret=False):
    @functools.partial(
        pl.kernel,
        out_type=jax.ShapeDtypeStruct((NW, HIST), jnp.float32),
        mesh=_MESH,
        scratch_types=[
            pltpu.VMEM((GRP, CHUNK), jnp.int32),
            pltpu.VMEM((HIST,), jnp.float32),
        ],
        compiler_params=_SC_PARAMS,
        interpret=interpret,
    )
    def deg_kernel(dst_hbm, pdeg_hbm, dstv, hist):
        c = lax.axis_index("c")
        s = lax.axis_index("s")
        w = s * NC + c

        def zacc(i, _):
            hist[pl.ds(pl.multiple_of(i * 16, 16), 16)] = _Z16F()
            return 0
        lax.fori_loop(0, HIST // 16, zacc, 0)
        ones16 = jnp.ones((16,), jnp.float32)

        pltpu.sync_copy(dst_hbm.at[pl.ds(w * GRP, GRP)], dstv)

        def row(r, _):
            for u in range(CHUNK // 16):
                d = dstv[r, pl.ds(u * 16, 16)]
                plsc.addupdate_scatter(hist, [jnp.minimum(d, HDUMP)], ones16)
            return 0
        lax.fori_loop(0, GRP, row, 0)

        pltpu.sync_copy(hist, pdeg_hbm.at[w])

    return deg_kernel


def _build_msg(interpret=False):
    @functools.partial(
        pl.kernel,
        out_type=jax.ShapeDtypeStruct((NW * WIN, FEAT), jnp.float32),
        mesh=_MESH,
        scratch_types=[
            pltpu.VMEM((GRP, CHUNK), jnp.int32),
            pltpu.VMEM((GRP, CHUNK), jnp.int32),
            pltpu.VMEM((2 * CHUNK,), jnp.int32),
            pltpu.VMEM((2 * CHUNK,), jnp.int32),
            pltpu.VMEM((CHUNK, FEAT), jnp.float32),
            pltpu.VMEM((ACC_ROWS, FEAT), jnp.float32),
            pltpu.SemaphoreType.DMA,
            pltpu.SemaphoreType.DMA,
        ],
        compiler_params=_SC_PARAMS,
        interpret=interpret,
    )
    def msg_kernel(hs_hbm, src_hbm, dst_hbm, agg_hbm,
                   srcv, dstv, sels, seld, rows, acc, sem, sem2):
        c = lax.axis_index("c")
        s = lax.axis_index("s")
        w = s * NC + c
        lo = w * WIN
        iota = lax.iota(jnp.int32, 16)

        def zacc(i, _):
            for u in range(FEAT // 16):
                acc[i, pl.ds(u * 16, 16)] = _Z16F()
            return 0
        lax.fori_loop(0, ACC_ROWS, zacc, 0)
        for u in range(2 * CHUNK // 16):
            seld[pl.ds(u * 16, 16)] = jnp.full((16,), DUMP, jnp.int32)
            sels[pl.ds(u * 16, 16)] = jnp.zeros((16,), jnp.int32)

        def flush(off):
            pltpu.async_copy(hs_hbm.at[sels.at[pl.ds(0, CHUNK)]], rows, sem).wait()

            def addgrp(g, _):
                vec = seld[pl.ds(pl.multiple_of(g * 16, 16), 16)]
                for l in range(16):
                    dj = jnp.sum(jnp.where(iota == l, vec, 0))
                    j = g * 16 + l
                    for u in range(FEAT // 16):
                        plsc.addupdate(acc.at[dj, pl.ds(u * 16, 16)],
                                       rows[j, pl.ds(u * 16, 16)])
                return 0
            lax.fori_loop(0, CHUNK // 16, addgrp, 0)
            for u in range(CHUNK // 16):
                seld[pl.ds(u * 16, 16)] = seld[pl.ds(CHUNK + u * 16, 16)]
                sels[pl.ds(u * 16, 16)] = sels[pl.ds(CHUNK + u * 16, 16)]
                seld[pl.ds(CHUNK + u * 16, 16)] = jnp.full((16,), DUMP, jnp.int32)
            return off - CHUNK

        def grp(g, off):
            h1 = pltpu.async_copy(src_hbm.at[pl.ds(g * GRP, GRP)], srcv, sem)
            h2 = pltpu.async_copy(dst_hbm.at[pl.ds(g * GRP, GRP)], dstv, sem2)
            h1.wait()
            h2.wait()

            def row(r, off):
                for u in range(CHUNK // 16):
                    d = dstv[r, pl.ds(u * 16, 16)]
                    sv = srcv[r, pl.ds(u * 16, 16)]
                    m = (d >= lo) & (d < lo + WIN)
                    plsc.store_compressed(seld.at[pl.ds(off, 16)], d - lo, mask=m)
                    plsc.store_compressed(sels.at[pl.ds(off, 16)], sv, mask=m)
                    off = off + jnp.sum(m.astype(jnp.int32))
                return lax.cond(off >= CHUNK, flush, lambda o: o, off)
            return lax.fori_loop(0, GRP, row, off)
        off = lax.fori_loop(0, NGRP, grp, jnp.int32(0))
        off = lax.cond(off > 0, lambda o: flush(o), lambda o: o, off)

        pltpu.sync_copy(acc.at[pl.ds(0, WIN)],
                        agg_hbm.at[pl.ds(w * WIN, WIN)])

    return msg_kernel


_ROW_BLK = 2000
_GRID = N_NODES // _ROW_BLK


def _gcn1_body(x_ref, w_ref, deg_ref, hs_ref):
    deg = jnp.sum(deg_ref[...], axis=1, keepdims=True)
    dinv = lax.rsqrt(deg + 1.0)
    h = jnp.dot(x_ref[...], w_ref[...], preferred_element_type=jnp.float32)
    hs_ref[...] = h * dinv


def _build_gcn1(interpret=False):
    return pl.pallas_call(
        _gcn1_body,
        grid=(_GRID,),
        in_specs=[
            pl.BlockSpec((_ROW_BLK, FEAT), lambda i: (i, 0)),
            pl.BlockSpec((FEAT, FEAT), lambda i: (0, 0)),
            pl.BlockSpec((_ROW_BLK, NW), lambda i: (i, 0)),
        ],
        out_specs=pl.BlockSpec((_ROW_BLK, FEAT), lambda i: (i, 0)),
        out_shape=jax.ShapeDtypeStruct((N_NODES, FEAT), jnp.float32),
        interpret=interpret,
    )


def _out_body(agg_ref, hs_ref, deg_ref, bg_ref, wfc_ref, bfc_ref, o_ref):
    deg = jnp.sum(deg_ref[...], axis=1, keepdims=True)
    dinv = lax.rsqrt(deg + 1.0)
    out = dinv * (agg_ref[...] + hs_ref[...]) + bg_ref[...]
    h2 = jnp.maximum(out, 0.0)
    lp = jnp.dot(h2, wfc_ref[...], preferred_element_type=jnp.float32) + bfc_ref[...]
    l0 = lp[:, 0:1]
    l1 = lp[:, 1:2]
    p0 = 1.0 / (1.0 + jnp.exp(l1 - l0))
    p1 = 1.0 / (1.0 + jnp.exp(l0 - l1))
    o_ref[...] = jnp.concatenate([p0, p1], axis=1)


def _build_out(interpret=False):
    return pl.pallas_call(
        _out_body,
        grid=(_GRID,),
        in_specs=[
            pl.BlockSpec((_ROW_BLK, FEAT), lambda i: (i, 0)),
            pl.BlockSpec((_ROW_BLK, FEAT), lambda i: (i, 0)),
            pl.BlockSpec((_ROW_BLK, NW), lambda i: (i, 0)),
            pl.BlockSpec((1, FEAT), lambda i: (0, 0)),
            pl.BlockSpec((FEAT, 128), lambda i: (0, 0)),
            pl.BlockSpec((1, 128), lambda i: (0, 0)),
        ],
        out_specs=pl.BlockSpec((_ROW_BLK, OUT_C), lambda i: (i, 0)),
        out_shape=jax.ShapeDtypeStruct((N_NODES, OUT_C), jnp.float32),
        interpret=interpret,
    )


_deg_kernel = _build_deg()
_msg_kernel = _build_msg()
_gcn1_kernel = _build_gcn1()
_out_kernel = _build_out()


def kernel(x, edge_index, W_gcn, b_gcn, W_fc, b_fc):
    src = edge_index[0].astype(jnp.int32)
    dst = edge_index[1].astype(jnp.int32)
    pad = E_PAD - N_EDGES
    src2d = jnp.concatenate([src, jnp.zeros((pad,), jnp.int32)]).reshape(ROWS2D, CHUNK)
    dst2d = jnp.concatenate([dst, jnp.full((pad,), PAD_DST, jnp.int32)]).reshape(ROWS2D, CHUNK)

    pdeg = _deg_kernel(dst2d)
    deg = pdeg[:, :N_NODES].T
    x2 = x.reshape(N_NODES, FEAT)
    hs = _gcn1_kernel(x2, W_gcn, deg)
    agg = _msg_kernel(hs, src2d, dst2d)

    wfc_p = jnp.pad(W_fc, ((0, 0), (0, 128 - OUT_C)))
    bfc_p = jnp.pad(b_fc, (0, 128 - OUT_C)).reshape(1, 128)
    probs = _out_kernel(agg, hs, deg, b_gcn.reshape(1, FEAT), wfc_p, bfc_p)
    return probs.reshape(1, N_NODES, OUT_C)


# speedup vs baseline: 7.4742x; 1.1776x over previous
import functools

import jax
import jax.numpy as jnp
from jax import lax
from jax.experimental import pallas as pl
from jax.experimental.pallas import tpu as pltpu
from jax.experimental.pallas import tpu_sc as plsc

N_NODES = 10000
FEAT = 256
N_EDGES = 160000
OUT_C = 2

NC = 2
NS = 16
NW = NC * NS
CHUNK = 128
E_PAD = 163840
ROWS2D = E_PAD // CHUNK
GRP = 40
NGRP = ROWS2D // GRP
PAD_DST = 1 << 29

FL = 64
WIN = 320
DUMP = WIN
ACC_ROWS = 328
HIST = 10368
HDUMP = 10240

_MESH = plsc.VectorSubcoreMesh(
    core_axis_name="c", subcore_axis_name="s", num_cores=NC, num_subcores=NS)

_Z16F = functools.partial(jnp.zeros, (16,), jnp.float32)
_SC_PARAMS = pltpu.CompilerParams(needs_layout_passes=False)


def _build_deg(interpret=False):
    @functools.partial(
        pl.kernel,
        out_type=jax.ShapeDtypeStruct((NW, HIST), jnp.float32),
        mesh=_MESH,
        scratch_types=[
            pltpu.VMEM((GRP, CHUNK), jnp.int32),
            pltpu.VMEM((HIST,), jnp.float32),
        ],
        compiler_params=_SC_PARAMS,
        interpret=interpret,
    )
    def deg_kernel(dst_hbm, pdeg_hbm, dstv, hist):
        c = lax.axis_index("c")
        s = lax.axis_index("s")
        w = s * NC + c

        def zacc(i, _):
            hist[pl.ds(pl.multiple_of(i * 16, 16), 16)] = _Z16F()
            return 0
        lax.fori_loop(0, HIST // 16, zacc, 0)
        ones16 = jnp.ones((16,), jnp.float32)

        pltpu.sync_copy(dst_hbm.at[pl.ds(w * GRP, GRP)], dstv)

        def row(r, _):
            for u in range(CHUNK // 16):
                d = dstv[r, pl.ds(u * 16, 16)]
                plsc.addupdate_scatter(hist, [jnp.minimum(d, HDUMP)], ones16)
            return 0
        lax.fori_loop(0, GRP, row, 0)

        pltpu.sync_copy(hist, pdeg_hbm.at[w])

    return deg_kernel


def _build_msg(interpret=False):
    @functools.partial(
        pl.kernel,
        out_type=jax.ShapeDtypeStruct((NW * WIN, FEAT), jnp.float32),
        mesh=_MESH,
        scratch_types=[
            pltpu.VMEM((GRP, CHUNK), jnp.int32),
            pltpu.VMEM((GRP, CHUNK), jnp.int32),
            pltpu.VMEM((2 * CHUNK,), jnp.int32),
            pltpu.VMEM((2 * CHUNK,), jnp.int32),
            pltpu.VMEM((2, FL), jnp.int32),
            pltpu.VMEM((2, FL), jnp.int32),
            pltpu.VMEM((2 * FL, FEAT), jnp.float32),
            pltpu.VMEM((ACC_ROWS, FEAT), jnp.float32),
            pltpu.SemaphoreType.DMA,
            pltpu.SemaphoreType.DMA,
            pltpu.SemaphoreType.DMA,
            pltpu.SemaphoreType.DMA,
        ],
        compiler_params=_SC_PARAMS,
        interpret=interpret,
    )
    def msg_kernel(hs_hbm, src_hbm, dst_hbm, agg_hbm,
                   srcv, dstv, sels, seld, psels, pseld, rows, acc,
                   semA, semB, semC, semD):
        c = lax.axis_index("c")
        s = lax.axis_index("s")
        w = s * NC + c
        lo = w * WIN
        iota = lax.iota(jnp.int32, 16)

        def zacc(i, _):
            for u in range(FEAT // 16):
                acc[i, pl.ds(u * 16, 16)] = _Z16F()
            return 0
        lax.fori_loop(0, ACC_ROWS, zacc, 0)
        for u in range(2 * CHUNK // 16):
            seld[pl.ds(u * 16, 16)] = jnp.full((16,), DUMP, jnp.int32)
            sels[pl.ds(u * 16, 16)] = jnp.zeros((16,), jnp.int32)

        def drain(q):
            @pl.when(q == 0)
            def _():
                pltpu.make_async_copy(hs_hbm.at[pl.ds(0, FL)],
                                      rows.at[pl.ds(0, FL)], semA).wait()

            @pl.when(q == 1)
            def _():
                pltpu.make_async_copy(hs_hbm.at[pl.ds(0, FL)],
                                      rows.at[pl.ds(FL, FL)], semB).wait()

        def addhalf(q):
            def addgrp(g, _):
                vec = pseld[q, pl.ds(pl.multiple_of(g * 16, 16), 16)]
                for l in range(16):
                    dj = jnp.sum(jnp.where(iota == l, vec, 0))
                    j = q * FL + g * 16 + l
                    for u in range(FEAT // 16):
                        plsc.addupdate(acc.at[dj, pl.ds(u * 16, 16)],
                                       rows[j, pl.ds(u * 16, 16)])
                return 0
            lax.fori_loop(0, FL // 16, addgrp, 0)

        def flush(state):
            off, p, pend = state
            for u in range(FL // 16):
                psels[p, pl.ds(u * 16, 16)] = sels[pl.ds(u * 16, 16)]
                pseld[p, pl.ds(u * 16, 16)] = seld[pl.ds(u * 16, 16)]

            @pl.when(p == 0)
            def _():
                pltpu.async_copy(hs_hbm.at[psels.at[0]],
                                 rows.at[pl.ds(0, FL)], semA)

            @pl.when(p == 1)
            def _():
                pltpu.async_copy(hs_hbm.at[psels.at[1]],
                                 rows.at[pl.ds(FL, FL)], semB)

            for u in range((2 * CHUNK - FL) // 16):
                seld[pl.ds(u * 16, 16)] = seld[pl.ds(FL + u * 16, 16)]
                sels[pl.ds(u * 16, 16)] = sels[pl.ds(FL + u * 16, 16)]
            for u in range(FL // 16):
                seld[pl.ds(2 * CHUNK - FL + u * 16, 16)] = (
                    jnp.full((16,), DUMP, jnp.int32))

            @pl.when(pend == 1)
            def _():
                drain(1 - p)
                addhalf(1 - p)
            return off - FL, 1 - p, jnp.int32(1)

        def grp(g, state):
            off, p, pend = state
            h1 = pltpu.async_copy(src_hbm.at[pl.ds(g * GRP, GRP)], srcv, semC)
            h2 = pltpu.async_copy(dst_hbm.at[pl.ds(g * GRP, GRP)], dstv, semD)
            h1.wait()
            h2.wait()

            def row(r, state):
                off, p, pend = state
                for u in range(CHUNK // 16):
                    d = dstv[r, pl.ds(u * 16, 16)]
                    sv = srcv[r, pl.ds(u * 16, 16)]
                    m = (d >= lo) & (d < lo + WIN)
                    plsc.store_compressed(seld.at[pl.ds(off, 16)], d - lo, mask=m)
                    plsc.store_compressed(sels.at[pl.ds(off, 16)], sv, mask=m)
                    off = off + jnp.sum(m.astype(jnp.int32))
                state = (off, p, pend)
                state = lax.cond(off >= FL, flush, lambda st: st, state)
                state = lax.cond(state[0] >= FL, flush, lambda st: st, state)
                return state
            return lax.fori_loop(0, GRP, row, (off, p, pend))
        state = lax.fori_loop(0, NGRP, grp,
                              (jnp.int32(0), jnp.int32(0), jnp.int32(0)))
        state = lax.cond(state[0] > 0, flush, lambda st: st, state)
        off, p, pend = state

        @pl.when(pend == 1)
        def _():
            drain(1 - p)
            addhalf(1 - p)

        pltpu.sync_copy(acc.at[pl.ds(0, WIN)],
                        agg_hbm.at[pl.ds(w * WIN, WIN)])

    return msg_kernel


_ROW_BLK = 2000
_GRID = N_NODES // _ROW_BLK


def _gcn1_body(x_ref, w_ref, deg_ref, hs_ref):
    deg = jnp.sum(deg_ref[...], axis=1, keepdims=True)
    dinv = lax.rsqrt(deg + 1.0)
    h = jnp.dot(x_ref[...], w_ref[...], preferred_element_type=jnp.float32)
    hs_ref[...] = h * dinv


def _build_gcn1(interpret=False):
    return pl.pallas_call(
        _gcn1_body,
        grid=(_GRID,),
        in_specs=[
            pl.BlockSpec((_ROW_BLK, FEAT), lambda i: (i, 0)),
            pl.BlockSpec((FEAT, FEAT), lambda i: (0, 0)),
            pl.BlockSpec((_ROW_BLK, NW), lambda i: (i, 0)),
        ],
        out_specs=pl.BlockSpec((_ROW_BLK, FEAT), lambda i: (i, 0)),
        out_shape=jax.ShapeDtypeStruct((N_NODES, FEAT), jnp.float32),
        interpret=interpret,
    )


def _out_body(agg_ref, hs_ref, deg_ref, bg_ref, wfc_ref, bfc_ref, o_ref):
    deg = jnp.sum(deg_ref[...], axis=1, keepdims=True)
    dinv = lax.rsqrt(deg + 1.0)
    out = dinv * (agg_ref[...] + hs_ref[...]) + bg_ref[...]
    h2 = jnp.maximum(out, 0.0)
    lp = jnp.dot(h2, wfc_ref[...], preferred_element_type=jnp.float32) + bfc_ref[...]
    l0 = lp[:, 0:1]
    l1 = lp[:, 1:2]
    p0 = 1.0 / (1.0 + jnp.exp(l1 - l0))
    p1 = 1.0 / (1.0 + jnp.exp(l0 - l1))
    o_ref[...] = jnp.concatenate([p0, p1], axis=1)


def _build_out(interpret=False):
    return pl.pallas_call(
        _out_body,
        grid=(_GRID,),
        in_specs=[
            pl.BlockSpec((_ROW_BLK, FEAT), lambda i: (i, 0)),
            pl.BlockSpec((_ROW_BLK, FEAT), lambda i: (i, 0)),
            pl.BlockSpec((_ROW_BLK, NW), lambda i: (i, 0)),
            pl.BlockSpec((1, FEAT), lambda i: (0, 0)),
            pl.BlockSpec((FEAT, 128), lambda i: (0, 0)),
            pl.BlockSpec((1, 128), lambda i: (0, 0)),
        ],
        out_specs=pl.BlockSpec((_ROW_BLK, OUT_C), lambda i: (i, 0)),
        out_shape=jax.ShapeDtypeStruct((N_NODES, OUT_C), jnp.float32),
        interpret=interpret,
    )


_deg_kernel = _build_deg()
_msg_kernel = _build_msg()
_gcn1_kernel = _build_gcn1()
_out_kernel = _build_out()


def kernel(x, edge_index, W_gcn, b_gcn, W_fc, b_fc):
    src = edge_index[0].astype(jnp.int32)
    dst = edge_index[1].astype(jnp.int32)
    pad = E_PAD - N_EDGES
    src2d = jnp.concatenate([src, jnp.zeros((pad,), jnp.int32)]).reshape(ROWS2D, CHUNK)
    dst2d = jnp.concatenate([dst, jnp.full((pad,), PAD_DST, jnp.int32)]).reshape(ROWS2D, CHUNK)

    pdeg = _deg_kernel(dst2d)
    deg = pdeg[:, :N_NODES].T
    x2 = x.reshape(N_NODES, FEAT)
    hs = _gcn1_kernel(x2, W_gcn, deg)
    agg = _msg_kernel(hs, src2d, dst2d)

    wfc_p = jnp.pad(W_fc, ((0, 0), (0, 128 - OUT_C)))
    bfc_p = jnp.pad(b_fc, (0, 128 - OUT_C)).reshape(1, 128)
    probs = _out_kernel(agg, hs, deg, b_gcn.reshape(1, FEAT), wfc_p, bfc_p)
    return probs.reshape(1, N_NODES, OUT_C)
